# baseline (speedup 1.0000x reference)
import jax
import jax.numpy as jnp
from jax import lax
from jax.experimental import pallas as pl
from jax.experimental.pallas import tpu as pltpu
from jax.experimental.pallas import tpu_sc as plsc

D = 64
V = 100000
B = 16384
NC, NS, L = 2, 16, 16
NW = NC * NS
ROWS_PER_W = D // NW
CHUNK = 4096
NCHUNK = B // CHUNK


def _gather_kernel(idx_hbm, table_hbm, out_hbm,
                   idx_v, row_v, out0_v, out1_v, sem_idx, sem_row, sem_o0, sem_o1):
    wid = lax.axis_index("s") * NC + lax.axis_index("c")
    base_row = wid * ROWS_PER_W
    pltpu.sync_copy(idx_hbm, idx_v)

    def row_body(r, carry):
        row = base_row + r
        pltpu.sync_copy(table_hbm.at[row], row_v)

        def chunk_body(c, carry2):
            @plsc.parallel_loop(0, CHUNK, step=L, unroll=8)
            def body(k):
                iv = idx_v[pl.ds(c * CHUNK + k, L)]
                out0_v[pl.ds(k, L)] = plsc.load_gather(row_v, [iv])

            pltpu.sync_copy(out0_v, out_hbm.at[row, pl.ds(c * CHUNK, CHUNK)])
            return carry2

        lax.fori_loop(0, NCHUNK, chunk_body, 0)
        return carry

    lax.fori_loop(0, ROWS_PER_W, row_body, 0)


@jax.jit
def _run(task_id, mesa_parameters):
    mesh = plsc.VectorSubcoreMesh(core_axis_name="c", subcore_axis_name="s")
    return pl.kernel(
        _gather_kernel,
        out_type=jax.ShapeDtypeStruct((D, B), jnp.float32),
        mesh=mesh,
        scratch_types=[
            pltpu.VMEM((B,), jnp.int32),
            pltpu.VMEM((V,), jnp.float32),
            pltpu.VMEM((CHUNK,), jnp.float32),
            pltpu.VMEM((CHUNK,), jnp.float32),
            pltpu.SemaphoreType.DMA,
            pltpu.SemaphoreType.DMA,
            pltpu.SemaphoreType.DMA,
            pltpu.SemaphoreType.DMA,
        ],
        compiler_params=pltpu.CompilerParams(needs_layout_passes=False),
    )(task_id, mesa_parameters)


def kernel(task_id, mesa_parameters):
    return _run(task_id.astype(jnp.int32), mesa_parameters)


# speedup vs baseline: 1.1360x; 1.1360x over previous
import jax
import jax.numpy as jnp
from jax import lax
from jax.experimental import pallas as pl
from jax.experimental.pallas import tpu as pltpu
from jax.experimental.pallas import tpu_sc as plsc

D = 64
V = 100000
B = 16384
NC, NS, L = 2, 16, 16
NW = NC * NS
ROWS_PER_W = D // NW
CHUNK = 4096
NCHUNK = B // CHUNK


def _gather_kernel(idx_hbm, table_hbm, out_hbm,
                   idx_v, idx_sh, row_v, out0_v, out1_v,
                   sem_idx, sem_row, sem_o0, sem_o1):
    sid = lax.axis_index("s")
    wid = sid * NC + lax.axis_index("c")
    base_row = wid * ROWS_PER_W
    cp_row = pltpu.async_copy(table_hbm.at[base_row], row_v, sem_row)

    @pl.when(sid == 0)
    def _():
        pltpu.sync_copy(idx_hbm, idx_sh)

    plsc.subcore_barrier()
    pltpu.sync_copy(idx_sh, idx_v)
    cp_row.wait()

    out_cps = [None, None]
    out_sems = [sem_o0, sem_o1]
    out_bufs = [out0_v, out1_v]
    for r in range(ROWS_PER_W):
        row = base_row + r
        for c in range(NCHUNK):
            t = (r * NCHUNK + c) % 2
            if out_cps[t] is not None:
                out_cps[t].wait()
            buf = out_bufs[t]

            @plsc.parallel_loop(0, CHUNK, step=L, unroll=8)
            def body(k):
                iv = idx_v[pl.ds(c * CHUNK + k, L)]
                buf[pl.ds(k, L)] = plsc.load_gather(row_v, [iv])

            out_cps[t] = pltpu.async_copy(
                buf, out_hbm.at[row, pl.ds(c * CHUNK, CHUNK)], out_sems[t])
        if r + 1 < ROWS_PER_W:
            pltpu.async_copy(table_hbm.at[base_row + r + 1], row_v, sem_row).wait()
    for cp in out_cps:
        cp.wait()


@jax.jit
def _run(task_id, mesa_parameters):
    mesh = plsc.VectorSubcoreMesh(core_axis_name="c", subcore_axis_name="s")
    return pl.kernel(
        _gather_kernel,
        out_type=jax.ShapeDtypeStruct((D, B), jnp.float32),
        mesh=mesh,
        scratch_types=[
            pltpu.VMEM((B,), jnp.int32),
            pltpu.VMEM_SHARED((B,), jnp.int32),
            pltpu.VMEM((V,), jnp.float32),
            pltpu.VMEM((CHUNK,), jnp.float32),
            pltpu.VMEM((CHUNK,), jnp.float32),
            pltpu.SemaphoreType.DMA,
            pltpu.SemaphoreType.DMA,
            pltpu.SemaphoreType.DMA,
            pltpu.SemaphoreType.DMA,
        ],
        compiler_params=pltpu.CompilerParams(needs_layout_passes=False),
    )(task_id, mesa_parameters)


def kernel(task_id, mesa_parameters):
    return _run(task_id.astype(jnp.int32), mesa_parameters)
